# Initial kernel scaffold; baseline (speedup 1.0000x reference)
import jax
import jax.numpy as jnp
from jax.experimental import pallas as pl


def kernel(x, edge_index, Wz, bz, Lz, lbz, Wr, br, Lr, lbr, Wh, bh, Lh, lbh, att, Wl, bl):
    raise NotImplementedError("write your pallas kernel here")


# speedup vs baseline: 111.0631x; 111.0631x over previous
import functools

import jax
import jax.numpy as jnp
from jax.experimental import pallas as pl
from jax.experimental.pallas import tpu as pltpu
from jax.experimental.pallas import tpu_sc as plsc

N = 10000
NPAD = 10240
E = 320000
EPAD = 327680
ROWS = EPAD // 128
RA = ROWS // 32
RC = ROWS // 16
F = 384
CW = 128
NCH = 3
NBUF = 2
WROWS = RA // 2
KD = 8
RPT = NPAD // 16

_SC_MESH = plsc.VectorSubcoreMesh(
    core_axis_name="c", subcore_axis_name="s", num_cores=2, num_subcores=16)


@functools.partial(
    pl.kernel,
    out_type=jax.ShapeDtypeStruct((2, NPAD, CW), jnp.float32),
    mesh=_SC_MESH,
    scratch_types=[
        pltpu.VMEM((RA, 128), jnp.int32),
        pltpu.VMEM((128, CW), jnp.float32),
        pltpu.SemaphoreType.DMA,
        pltpu.VMEM_SHARED((NPAD, CW), jnp.float32),
    ],
)
def _deg_kernel(dst_hbm, zeros8_hbm, ones_hbm, out_hbm, idx_v, ones_v, ssem,
                deg_sh):
    c = jax.lax.axis_index("c")
    s = jax.lax.axis_index("s")
    pltpu.sync_copy(zeros8_hbm.at[pl.ds(s * RPT, RPT)],
                    deg_sh.at[pl.ds(s * RPT, RPT)])
    pltpu.sync_copy(ones_hbm, ones_v)
    w = c * 16 + s
    pltpu.sync_copy(dst_hbm.at[pl.ds(w * RA, RA)], idx_v)
    plsc.subcore_barrier()

    def body(it, carry):
        jo = it * KD
        for b in range(KD):
            pltpu.async_copy(ones_v, deg_sh.at[idx_v.at[jo + b]], ssem,
                             add=True)
        for b in range(KD):
            pltpu.make_async_copy(ones_v, deg_sh.at[idx_v.at[jo]],
                                  ssem).wait()
        return carry

    jax.lax.fori_loop(0, RA // KD, body, 0)
    plsc.subcore_barrier()

    @pl.when(c == 0)
    def _():
        pltpu.sync_copy(deg_sh.at[pl.ds(s * RPT, RPT)],
                        out_hbm.at[0, pl.ds(s * RPT, RPT)])

    @pl.when(c == 1)
    def _():
        pltpu.sync_copy(deg_sh.at[pl.ds(s * RPT, RPT)],
                        out_hbm.at[1, pl.ds(s * RPT, RPT)])


@functools.partial(
    pl.kernel,
    out_type=jax.ShapeDtypeStruct((2 * NCH, NPAD, CW), jnp.float32),
    mesh=_SC_MESH,
    scratch_types=[
        pltpu.VMEM((WROWS, 128), jnp.int32),
        pltpu.VMEM((WROWS, 128), jnp.int32),
        pltpu.VMEM((NBUF, 128, CW), jnp.float32),
        pltpu.SemaphoreType.DMA,
        pltpu.SemaphoreType.DMA,
        pltpu.SemaphoreType.DMA,
        pltpu.SemaphoreType.DMA,
        pltpu.VMEM_SHARED((NPAD, CW), jnp.float32),
    ],
)
def _agg_kernel(xs0_hbm, xs1_hbm, xs2_hbm, src_hbm, dst_hbm, zeros_hbm,
                out_hbm, swin, dwin, rows_v, g0, g1, s0, s1, agg_sh):
    gsem = (g0, g1)
    ssem = (s0, s1)
    c = jax.lax.axis_index("c")
    s = jax.lax.axis_index("s")
    w = c * 16 + s

    for ch, xs_hbm in enumerate((xs0_hbm, xs1_hbm, xs2_hbm)):
        pltpu.sync_copy(zeros_hbm.at[pl.ds(s * RPT, RPT)],
                        agg_sh.at[pl.ds(s * RPT, RPT)])
        pltpu.sync_copy(src_hbm.at[pl.ds(w * RA, WROWS)], swin)
        pltpu.sync_copy(dst_hbm.at[pl.ds(w * RA, WROWS)], dwin)
        plsc.subcore_barrier()

        for b in range(NBUF):
            pltpu.async_copy(xs_hbm.at[swin.at[b]], rows_v.at[b], gsem[b])

        def body(it, carry):
            jo = it * NBUF
            woff = jnp.where(jo >= WROWS, WROWS, 0)
            for b in range(NBUF):
                pltpu.make_async_copy(xs_hbm.at[swin.at[0]], rows_v.at[b],
                                      gsem[b]).wait()
                pltpu.async_copy(rows_v.at[b],
                                 agg_sh.at[dwin.at[jo + b - woff]],
                                 ssem[b], add=True)

            for b in range(NBUF):
                pltpu.make_async_copy(rows_v.at[b], agg_sh.at[dwin.at[0]],
                                      ssem[b]).wait()

            @pl.when(it == WROWS // NBUF - 1)
            def _():
                pltpu.sync_copy(src_hbm.at[pl.ds(w * RA + WROWS, WROWS)],
                                swin)
                pltpu.sync_copy(dst_hbm.at[pl.ds(w * RA + WROWS, WROWS)],
                                dwin)

            for b in range(NBUF):
                jn = jnp.minimum(jo + NBUF + b, RA - 1)
                jnw = jn - jnp.where(jn >= WROWS, WROWS, 0)
                pltpu.async_copy(xs_hbm.at[swin.at[jnw]], rows_v.at[b],
                                 gsem[b])
            return carry

        jax.lax.fori_loop(0, RA // NBUF, body, 0)

        for b in range(NBUF):
            pltpu.make_async_copy(xs_hbm.at[swin.at[0]], rows_v.at[b],
                                  gsem[b]).wait()
        plsc.subcore_barrier()

        @pl.when(c == 0)
        def _():
            pltpu.sync_copy(agg_sh.at[pl.ds(s * RPT, RPT)],
                            out_hbm.at[ch, pl.ds(s * RPT, RPT)])

        @pl.when(c == 1)
        def _():
            pltpu.sync_copy(agg_sh.at[pl.ds(s * RPT, RPT)],
                            out_hbm.at[NCH + ch, pl.ds(s * RPT, RPT)])


_BN = 2560


def _scale_body(xt_ref, deg_ref, xs0_ref, xs1_ref, xs2_ref):
    deg = deg_ref[0, :, 0:1] + deg_ref[1, :, 0:1] + 1.0
    dinv = jax.lax.rsqrt(deg)
    xs = xt_ref[...] * dinv
    xs0_ref[...] = xs[:, 0 * CW:1 * CW]
    xs1_ref[...] = xs[:, 1 * CW:2 * CW]
    xs2_ref[...] = xs[:, 2 * CW:3 * CW]


def _scale_kernel(xt, deg8):
    return pl.pallas_call(
        _scale_body,
        grid=(NPAD // _BN,),
        in_specs=[
            pl.BlockSpec((_BN, F), lambda i: (i, 0)),
            pl.BlockSpec((2, _BN, CW), lambda i: (0, i, 0)),
        ],
        out_specs=[
            pl.BlockSpec((_BN, CW), lambda i: (i, 0)),
            pl.BlockSpec((_BN, CW), lambda i: (i, 0)),
            pl.BlockSpec((_BN, CW), lambda i: (i, 0)),
        ],
        out_shape=[
            jax.ShapeDtypeStruct((NPAD, CW), jnp.float32),
            jax.ShapeDtypeStruct((NPAD, CW), jnp.float32),
            jax.ShapeDtypeStruct((NPAD, CW), jnp.float32),
        ],
    )(xt, deg8)


def _prep_body(wz_ref, lz_ref, bz_ref, lbz_ref, wh_ref, lh_ref, bh_ref,
               lbh_ref, att_ref, az_ref, ah_ref, czh_ref, probs_ref):
    lz = lz_ref[:128, :]
    lh = lh_ref[:128, :]
    az_ref[...] = jnp.dot(wz_ref[...], lz, preferred_element_type=jnp.float32)
    ah_ref[...] = jnp.dot(wh_ref[...], lh, preferred_element_type=jnp.float32)
    czh_ref[0:1, :] = jnp.dot(bz_ref[...], lz,
                              preferred_element_type=jnp.float32) + lbz_ref[...]
    czh_ref[1:2, :] = jnp.dot(bh_ref[...], lh,
                              preferred_element_type=jnp.float32) + lbh_ref[...]
    a = att_ref[...]
    m = jnp.max(a, axis=1, keepdims=True)
    e = jnp.exp(a - m)
    probs_ref[...] = e / jnp.sum(e, axis=1, keepdims=True)


def _prep_kernel(Wz, Lz, bz, lbz, Wh, Lh, bh, lbh, att):
    return pl.pallas_call(
        _prep_body,
        out_shape=[
            jax.ShapeDtypeStruct((32, 128), jnp.float32),
            jax.ShapeDtypeStruct((32, 128), jnp.float32),
            jax.ShapeDtypeStruct((2, 128), jnp.float32),
            jax.ShapeDtypeStruct((1, 12), jnp.float32),
        ],
    )(Wz, Lz, bz, lbz, Wh, Lh, bh, lbh, att)


def _final_body(agg_ref, xs0_ref, xs1_ref, xs2_ref, deg_ref, az_ref, ah_ref,
                czh_ref, probs_ref, wl_ref, bl_ref, out_ref):
    deg = deg_ref[0, :, 0:1] + deg_ref[1, :, 0:1] + 1.0
    dinv = jax.lax.rsqrt(deg)
    Az = az_ref[...]
    Ah = ah_ref[...]
    cz = czh_ref[0:1, :]
    ch = czh_ref[1:2, :]
    xs_refs = (xs0_ref, xs1_ref, xs2_ref)
    acc = jnp.zeros((_BN, 128), jnp.float32)
    for t in range(12):
        chk, tl = divmod(t, 4)
        off = tl * 32
        a = (agg_ref[chk, :, off:off + 32]
             + agg_ref[NCH + chk, :, off:off + 32]
             + xs_refs[chk][:, off:off + 32]) * dinv
        z = jax.nn.sigmoid(
            jnp.dot(a, Az, preferred_element_type=jnp.float32) + cz)
        hh = jnp.tanh(
            jnp.dot(a, Ah, preferred_element_type=jnp.float32) + ch)
        acc = acc + probs_ref[0, t] * ((1.0 - z) * hh)
    r = jnp.dot(jax.nn.relu(acc), wl_ref[...],
                preferred_element_type=jnp.float32) + bl_ref[...]
    m = jnp.max(r, axis=1, keepdims=True)
    e = jnp.exp(r - m)
    out_ref[...] = e / jnp.sum(e, axis=1, keepdims=True)


def _final_kernel(agg, xs0, xs1, xs2, deg8, Az, Ah, czh, probs, Wl, bl):
    return pl.pallas_call(
        _final_body,
        grid=(NPAD // _BN,),
        in_specs=[
            pl.BlockSpec((2 * NCH, _BN, CW), lambda i: (0, i, 0)),
            pl.BlockSpec((_BN, CW), lambda i: (i, 0)),
            pl.BlockSpec((_BN, CW), lambda i: (i, 0)),
            pl.BlockSpec((_BN, CW), lambda i: (i, 0)),
            pl.BlockSpec((2, _BN, CW), lambda i: (0, i, 0)),
            pl.BlockSpec((32, 128), lambda i: (0, 0)),
            pl.BlockSpec((32, 128), lambda i: (0, 0)),
            pl.BlockSpec((2, 128), lambda i: (0, 0)),
            pl.BlockSpec(memory_space=pltpu.SMEM),
            pl.BlockSpec((128, 128), lambda i: (0, 0)),
            pl.BlockSpec((1, 128), lambda i: (0, 0)),
        ],
        out_specs=pl.BlockSpec((_BN, 128), lambda i: (i, 0)),
        out_shape=jax.ShapeDtypeStruct((NPAD, 128), jnp.float32),
    )(agg, xs0, xs1, xs2, deg8, Az, Ah, czh, probs, Wl, bl)


def kernel(x, edge_index, Wz, bz, Lz, lbz, Wr, br, Lr, lbr, Wh, bh, Lh, lbh,
           att, Wl, bl):
    xt = jnp.pad(x.transpose(0, 2, 1).reshape(N, F), ((0, NPAD - N), (0, 0)))
    pad = N + jnp.arange(EPAD - E, dtype=jnp.int32) % (NPAD - N)
    srcp = jnp.concatenate([edge_index[0].astype(jnp.int32), pad]).reshape(ROWS, 128)
    dstp = jnp.concatenate([edge_index[1].astype(jnp.int32), pad]).reshape(ROWS, 128)
    zerosF = jnp.zeros((NPAD, CW), jnp.float32)
    onesF = jnp.ones((128, CW), jnp.float32)

    deg8 = _deg_kernel(dstp, zerosF, onesF)
    xs0, xs1, xs2 = _scale_kernel(xt, deg8)
    agg = _agg_kernel(xs0, xs1, xs2, srcp, dstp, zerosF)
    Az, Ah, czh, probs = _prep_kernel(
        Wz, Lz, bz.reshape(1, -1), lbz.reshape(1, -1),
        Wh, Lh, bh.reshape(1, -1), lbh.reshape(1, -1), att.reshape(1, -1))
    wl_pad = jnp.pad(Wl, ((0, 0), (0, 124)))
    bl_pad = jnp.pad(bl.reshape(1, -1), ((0, 0), (0, 124)),
                     constant_values=-1e30)
    out = _final_kernel(agg, xs0, xs1, xs2, deg8, Az, Ah, czh, probs,
                        wl_pad, bl_pad)
    return out[:N, :4]
